# baseline (speedup 1.0000x reference)
import functools

import jax
import jax.numpy as jnp
from jax import lax
from jax.experimental import pallas as pl
from jax.experimental.pallas import tpu as pltpu
from jax.experimental.pallas import tpu_sc as plsc

NC = 2
NS = 16
NW = NC * NS

B = 16384
S = 50
D = 128
ROWS = B * S
CHUNK = 80
K = 4
NB = 2 * K
PER_W = ROWS // NW
N_CHUNKS = PER_W // CHUNK
N_GROUPS = N_CHUNKS // NB
assert N_CHUNKS % NB == 0 and N_GROUPS >= 3

_mesh = plsc.VectorSubcoreMesh(
    core_axis_name="c", subcore_axis_name="s", num_cores=NC, num_subcores=NS
)


@functools.partial(
    pl.kernel,
    out_type=jax.ShapeDtypeStruct((ROWS, D), jnp.float32),
    mesh=_mesh,
    scratch_types=[
        pltpu.VMEM((N_CHUNKS, CHUNK), jnp.int32),
        [pltpu.VMEM((CHUNK, D), jnp.float32) for _ in range(NB)],
        [pltpu.SemaphoreType.DMA for _ in range(NB)],
        [pltpu.SemaphoreType.DMA for _ in range(NB)],
    ],
)
def _sc_gather(idx_hbm, table_hbm, out_hbm, idx_v, rows, gsems, wsems):
    wid = lax.axis_index("s") * NC + lax.axis_index("c")
    chunk0 = wid * N_CHUNKS
    row0 = chunk0 * CHUNK

    pltpu.sync_copy(idx_hbm.at[pl.ds(chunk0, N_CHUNKS)], idx_v)

    def g_fire(j, b):
        pltpu.async_copy(table_hbm.at[idx_v.at[j]], rows[b], gsems[b])

    def g_wait(b):
        pltpu.make_async_copy(table_hbm.at[idx_v.at[0]], rows[b], gsems[b]).wait()

    def w_fire(j, b):
        pltpu.async_copy(rows[b], out_hbm.at[pl.ds(row0 + j * CHUNK, CHUNK)],
                         wsems[b])

    def w_wait(b):
        pltpu.make_async_copy(rows[b], out_hbm.at[pl.ds(row0, CHUNK)],
                              wsems[b]).wait()

    for b in range(K):
        g_fire(b, b)

    for b in range(NB):
        if b >= K:
            w_wait((b - K) % NB)
        g_fire(b + K, (b + K) % NB)
        g_wait(b)
        w_fire(b, b)

    def group(g, carry):
        for b in range(NB):
            s = g * NB + b
            w_wait((b - K) % NB)
            g_fire(s + K, (b + K) % NB)
            g_wait(b)
            w_fire(s, b)
        return carry

    lax.fori_loop(1, N_GROUPS - 1, group, 0, unroll=False)

    s0 = (N_GROUPS - 1) * NB
    for b in range(NB):
        w_wait((b - K) % NB)
        if b < K:
            g_fire(s0 + b + K, (b + K) % NB)
        g_wait(b)
        w_fire(s0 + b, b)
    for b in range(K, NB):
        w_wait(b)


def kernel(inputs, embeddings):
    idx = inputs.T.astype(jnp.int32).reshape(ROWS // CHUNK, CHUNK)
    out = _sc_gather(idx, embeddings)
    return out.reshape(S, B, D).transpose(1, 0, 2)


# speedup vs baseline: 1.0032x; 1.0032x over previous
import functools

import jax
import jax.numpy as jnp
from jax import lax
from jax.experimental import pallas as pl
from jax.experimental.pallas import tpu as pltpu
from jax.experimental.pallas import tpu_sc as plsc

NC = 2
NS = 16
NW = NC * NS

B = 16384
S = 50
D = 128
ROWS = B * S
CHUNK = 128
K = 2
NB = 2 * K
PER_W = ROWS // NW
N_CHUNKS = PER_W // CHUNK
N_GROUPS = N_CHUNKS // NB
assert N_CHUNKS % NB == 0 and N_GROUPS >= 3

_mesh = plsc.VectorSubcoreMesh(
    core_axis_name="c", subcore_axis_name="s", num_cores=NC, num_subcores=NS
)


@functools.partial(
    pl.kernel,
    out_type=jax.ShapeDtypeStruct((ROWS, D), jnp.float32),
    mesh=_mesh,
    scratch_types=[
        pltpu.VMEM((N_CHUNKS, CHUNK), jnp.int32),
        [pltpu.VMEM((CHUNK, D), jnp.float32) for _ in range(NB)],
        [pltpu.SemaphoreType.DMA for _ in range(NB)],
        [pltpu.SemaphoreType.DMA for _ in range(NB)],
    ],
)
def _sc_gather(idx_hbm, table_hbm, out_hbm, idx_v, rows, gsems, wsems):
    wid = lax.axis_index("s") * NC + lax.axis_index("c")
    chunk0 = wid * N_CHUNKS
    row0 = chunk0 * CHUNK

    pltpu.sync_copy(idx_hbm.at[pl.ds(chunk0, N_CHUNKS)], idx_v)

    def g_fire(j, b):
        pltpu.async_copy(table_hbm.at[idx_v.at[j]], rows[b], gsems[b])

    def g_wait(b):
        pltpu.make_async_copy(table_hbm.at[idx_v.at[0]], rows[b], gsems[b]).wait()

    def w_fire(j, b):
        pltpu.async_copy(rows[b], out_hbm.at[pl.ds(row0 + j * CHUNK, CHUNK)],
                         wsems[b])

    def w_wait(b):
        pltpu.make_async_copy(rows[b], out_hbm.at[pl.ds(row0, CHUNK)],
                              wsems[b]).wait()

    for b in range(K):
        g_fire(b, b)

    for b in range(NB):
        if b >= K:
            w_wait((b - K) % NB)
        g_fire(b + K, (b + K) % NB)
        g_wait(b)
        w_fire(b, b)

    def group(g, carry):
        for b in range(NB):
            s = g * NB + b
            w_wait((b - K) % NB)
            g_fire(s + K, (b + K) % NB)
            g_wait(b)
            w_fire(s, b)
        return carry

    lax.fori_loop(1, N_GROUPS - 1, group, 0, unroll=False)

    s0 = (N_GROUPS - 1) * NB
    for b in range(NB):
        w_wait((b - K) % NB)
        if b < K:
            g_fire(s0 + b + K, (b + K) % NB)
        g_wait(b)
        w_fire(s0 + b, b)
    for b in range(K, NB):
        w_wait(b)


def kernel(inputs, embeddings):
    idx = inputs.T.astype(jnp.int32).reshape(ROWS // CHUNK, CHUNK)
    out = _sc_gather(idx, embeddings)
    return out.reshape(S, B, D).transpose(1, 0, 2)
